# baseline (speedup 1.0000x reference)
import functools

import jax
import jax.numpy as jnp
from jax import lax
from jax.experimental import pallas as pl
from jax.experimental.pallas import tpu as pltpu
from jax.experimental.pallas import tpu_sc as plsc

NW = 32
LANES = 16
BIG = 512
GSUB = 128
R_MAX = 4.0
PREF = (2.0 / R_MAX) ** 0.5
PI = 3.14159265358979323846

SIN_C = (1.0000000e+00, -1.6666666e-01, 8.3333291e-03, -1.9840954e-04,
         2.7546378e-06, -2.4848902e-08, 1.4123680e-10)
COS_C = (1.0000000e+00, -5.0000000e-01, 4.1666668e-02, -1.3888883e-03,
         2.4801293e-05, -2.7548450e-07, 2.0728559e-09, -1.0172918e-11)


def _horner(coeffs, x):
    r = jnp.full((LANES,), coeffs[-1], dtype=jnp.float32)
    for c in reversed(coeffs[:-1]):
        r = r * x + c
    return r


def _edge_body(n_atoms, n_edges, pos8_hbm, cen_hbm, nei_hbm, at_hbm,
               t0_hbm, t1_hbm, out_hbm,
               at_v, t0_v, t1_v, cidx, nidx, posc, posn, outb,
               semi, semg, semo):
    nblk_tot = n_edges // BIG
    nvec = BIG // LANES

    wid = lax.axis_index("s") * 2 + lax.axis_index("c")
    nblk_w = (nblk_tot - wid + NW - 1) // NW

    pltpu.sync_copy(at_hbm, at_v)
    pltpu.sync_copy(t0_hbm, t0_v)
    pltpu.sync_copy(t1_hbm, t1_v)

    def slot_base(s):
        sc = jnp.minimum(s, nblk_w - 1)
        return pl.multiple_of((wid + NW * sc) * BIG, 8)

    def fire_idx(s, i4):
        base = slot_base(s)
        pltpu.async_copy(cen_hbm.at[pl.ds(base, BIG)], cidx.at[i4], semi.at[i4])
        pltpu.async_copy(nei_hbm.at[pl.ds(base, BIG)], nidx.at[i4], semi.at[i4])

    def wait_idx(i4):
        pltpu.make_async_copy(cen_hbm.at[pl.ds(0, BIG)], cidx.at[i4], semi.at[i4]).wait()
        pltpu.make_async_copy(nei_hbm.at[pl.ds(0, BIG)], nidx.at[i4], semi.at[i4]).wait()

    def fire_gather(i4, i2):
        for j in range(BIG // GSUB):
            sl = pl.ds(j * GSUB, GSUB)
            pltpu.async_copy(pos8_hbm.at[cidx.at[i4, sl]], posc.at[i2, sl, :], semg.at[i2])
            pltpu.async_copy(pos8_hbm.at[nidx.at[i4, sl]], posn.at[i2, sl, :], semg.at[i2])

    def wait_gather(i4, i2):
        pltpu.make_async_copy(pos8_hbm.at[cidx.at[i4]], posc.at[i2], semg.at[i2]).wait()
        pltpu.make_async_copy(pos8_hbm.at[nidx.at[i4]], posn.at[i2], semg.at[i2]).wait()

    def fire_out(s, i2):
        obase = slot_base(s) * 8
        for t1 in range(5):
            pltpu.async_copy(outb.at[i2, pl.ds(t1 * 4096, 4096)],
                             out_hbm.at[t1, pl.ds(obase, 4096)], semo.at[i2])

    def wait_out(i2):
        pltpu.make_async_copy(outb.at[i2], out_hbm.at[0, pl.ds(0, BIG * 40)],
                              semo.at[i2]).wait()

    iota = lax.iota(jnp.int32, LANES)
    zero_col = jnp.zeros((LANES,), jnp.int32)
    one_col = jnp.full((LANES,), 1, jnp.int32)
    two_col = jnp.full((LANES,), 2, jnp.int32)

    def compute_block(i4, i2):
        cidx_b, nidx_b = cidx.at[i4], nidx.at[i4]
        posc_b, posn_b = posc.at[i2], posn.at[i2]
        outb_b = outb.at[i2]

        @plsc.parallel_loop(0, nvec, unroll=4)
        def compute_vec(j):
            jb = j * LANES
            rows = iota + jb
            cvec = cidx_b[pl.ds(jb, LANES)]
            nvec_i = nidx_b[pl.ds(jb, LANES)]
            tidc = plsc.load_gather(at_v, [cvec])
            tidn = plsc.load_gather(at_v, [nvec_i])

            cx = plsc.load_gather(posc_b, [rows, zero_col])
            cy = plsc.load_gather(posc_b, [rows, one_col])
            cz = plsc.load_gather(posc_b, [rows, two_col])
            nx = plsc.load_gather(posn_b, [rows, zero_col])
            ny = plsc.load_gather(posn_b, [rows, one_col])
            nz = plsc.load_gather(posn_b, [rows, two_col])

            dx = nx - cx
            dy = ny - cy
            dz = nz - cz
            s = dx * dx + dy * dy + dz * dz

            ibits = plsc.bitcast(s, jnp.int32)
            yi = jnp.full((LANES,), 0x5F3759DF, jnp.int32) - lax.shift_right_arithmetic(ibits, 1)
            y = plsc.bitcast(yi, jnp.float32)
            hs = 0.5 * s
            for _ in range(2):
                y = y * (1.5 - hs * y * y)
            x = s * y

            th = (PI / R_MAX) * x
            t2 = th * th
            s1 = _horner(SIN_C, t2) * th
            c1 = _horner(COS_C, t2)
            two_c1 = c1 + c1

            r = (1.0 / R_MAX) * x
            r2 = r * r
            r6 = r2 * r2 * r2
            cut = 1.0 - r6 * (28.0 - 48.0 * r + 21.0 * r2)
            cut = jnp.where(s < R_MAX * R_MAX, cut, 0.0)

            pcut = (PREF * cut) * y

            obase = (lax.shift_left(lax.shift_right_logical(rows, 7), 10)
                     + (rows & 127))

            def st(k, val):
                plsc.store_scatter(outb_b,
                                   [obase + ((k // 8) * 4096 + (k % 8) * 128)],
                                   val)

            st(0, s1 * pcut)
            skm2 = s1
            skm1 = two_c1 * s1
            st(1, skm1 * pcut)
            for k in range(3, 9):
                sk = two_c1 * skm1 - skm2
                st(k - 1, sk * pcut)
                skm2, skm1 = skm1, sk

            tc16 = tidc * 16
            tn16 = tidn * 16
            for c in range(16):
                v0 = plsc.load_gather(t0_v, [tc16 + c])
                st(8 + c, v0 * cut)
                v1 = plsc.load_gather(t1_v, [tn16 + c])
                st(24 + c, v1 * cut)

    for i in range(4):
        fire_idx(i, i)
    wait_idx(0)
    fire_gather(0, 0)

    n_iter = (nblk_w + 3) // 4

    def pipe_iter(t, _):
        s0 = t * 4
        for i in range(4):
            s = s0 + i
            i2 = i % 2
            wait_idx((i + 1) % 4)
            fire_gather((i + 1) % 4, (i + 1) % 2)
            wait_gather(i, i2)

            @pl.when((s - 2 >= 0) & (s - 2 < nblk_w))
            def _():
                wait_out(i2)

            compute_block(i, i2)

            @pl.when(s < nblk_w)
            def _():
                fire_out(s, i2)

            fire_idx(s + 4, i)
        return ()

    lax.fori_loop(0, n_iter, pipe_iter, (), unroll=False)

    wait_gather(0, 0)
    for i in (1, 2, 3):
        wait_idx(i)
    last = n_iter * 4

    @pl.when(last - 2 < nblk_w)
    def _():
        wait_out(0)

    @pl.when(last - 1 < nblk_w)
    def _():
        wait_out(1)


def kernel(pos, edge_index, atom_types, type_embeddings):
    n_atoms = pos.shape[0]
    n_edges = edge_index.shape[1]

    pos8 = jnp.pad(pos.astype(jnp.float32), ((0, 0), (0, 5)))
    cen = edge_index[0].astype(jnp.int32)
    nei = edge_index[1].astype(jnp.int32)
    at32 = atom_types.astype(jnp.int32)
    t0 = type_embeddings[0].astype(jnp.float32).reshape(-1)
    t1 = type_embeddings[1].astype(jnp.float32).reshape(-1)

    mesh = plsc.VectorSubcoreMesh(core_axis_name="c", subcore_axis_name="s",
                                  num_cores=2, num_subcores=16)
    body = functools.partial(_edge_body, n_atoms, n_edges)
    out = pl.kernel(
        body,
        out_type=jax.ShapeDtypeStruct((5, n_edges * 8), jnp.float32),
        mesh=mesh,
        compiler_params=pltpu.CompilerParams(needs_layout_passes=False,
                                             use_tc_tiling_on_sc=False),
        scratch_types=[
            pltpu.VMEM((n_atoms,), jnp.int32),
            pltpu.VMEM((256,), jnp.float32),
            pltpu.VMEM((256,), jnp.float32),
            pltpu.VMEM((4, BIG), jnp.int32),
            pltpu.VMEM((4, BIG), jnp.int32),
            pltpu.VMEM((2, BIG, 8), jnp.float32),
            pltpu.VMEM((2, BIG, 8), jnp.float32),
            pltpu.VMEM((2, BIG * 40), jnp.float32),
            pltpu.SemaphoreType.DMA((4,)),
            pltpu.SemaphoreType.DMA((2,)),
            pltpu.SemaphoreType.DMA((2,)),
        ],
    )(pos8, cen, nei, at32, t0, t1)
    return (out.reshape(5, n_edges // 128, 8, 128)
               .transpose(1, 3, 0, 2).reshape(n_edges, 40))


# speedup vs baseline: 1.0123x; 1.0123x over previous
import functools

import jax
import jax.numpy as jnp
from jax import lax
from jax.experimental import pallas as pl
from jax.experimental.pallas import tpu as pltpu
from jax.experimental.pallas import tpu_sc as plsc

NW = 32
LANES = 16
BIG = 512
GSUB = 128
R_MAX = 4.0
PREF = (2.0 / R_MAX) ** 0.5
PI = 3.14159265358979323846

SIN_C = (1.0000000e+00, -1.6666666e-01, 8.3333291e-03, -1.9840954e-04,
         2.7546378e-06, -2.4848902e-08, 1.4123680e-10)
COS_C = (1.0000000e+00, -5.0000000e-01, 4.1666668e-02, -1.3888883e-03,
         2.4801293e-05, -2.7548450e-07, 2.0728559e-09, -1.0172918e-11)


def _horner(coeffs, x):
    r = jnp.full((LANES,), coeffs[-1], dtype=jnp.float32)
    for c in reversed(coeffs[:-1]):
        r = r * x + c
    return r


def _edge_body(n_atoms, n_edges, pos8_hbm, cen_hbm, nei_hbm, at_hbm,
               t0_hbm, t1_hbm, out_hbm,
               at_v, t0_v, t1_v, cidx, nidx, posc, posn, outb,
               semi, semg, semo):
    nblk_tot = n_edges // BIG
    nvec = BIG // LANES

    wid = lax.axis_index("s") * 2 + lax.axis_index("c")
    nblk_w = (nblk_tot - wid + NW - 1) // NW

    pltpu.sync_copy(at_hbm, at_v)
    pltpu.sync_copy(t0_hbm, t0_v)
    pltpu.sync_copy(t1_hbm, t1_v)

    def slot_base(s):
        sc = jnp.minimum(s, nblk_w - 1)
        return pl.multiple_of((wid + NW * sc) * BIG, 8)

    def fire_idx(s, i4):
        base = slot_base(s)
        pltpu.async_copy(cen_hbm.at[pl.ds(base, BIG)], cidx.at[i4], semi.at[i4])
        pltpu.async_copy(nei_hbm.at[pl.ds(base, BIG)], nidx.at[i4], semi.at[i4])

    def wait_idx(i4):
        pltpu.make_async_copy(cen_hbm.at[pl.ds(0, BIG)], cidx.at[i4], semi.at[i4]).wait()
        pltpu.make_async_copy(nei_hbm.at[pl.ds(0, BIG)], nidx.at[i4], semi.at[i4]).wait()

    def fire_gather(i4, i2):
        for j in range(BIG // GSUB):
            sl = pl.ds(j * GSUB, GSUB)
            pltpu.async_copy(pos8_hbm.at[cidx.at[i4, sl]], posc.at[i2, sl, :], semg.at[i2])
            pltpu.async_copy(pos8_hbm.at[nidx.at[i4, sl]], posn.at[i2, sl, :], semg.at[i2])

    def wait_gather(i4, i2):
        pltpu.make_async_copy(pos8_hbm.at[cidx.at[i4]], posc.at[i2], semg.at[i2]).wait()
        pltpu.make_async_copy(pos8_hbm.at[nidx.at[i4]], posn.at[i2], semg.at[i2]).wait()

    def fire_out(s, i2):
        obase = slot_base(s) * 8
        for t1 in range(5):
            pltpu.async_copy(outb.at[i2, pl.ds(t1 * 4096, 4096)],
                             out_hbm.at[t1, pl.ds(obase, 4096)], semo.at[i2])

    def wait_out(i2):
        pltpu.make_async_copy(outb.at[i2], out_hbm.at[0, pl.ds(0, BIG * 40)],
                              semo.at[i2]).wait()

    iota = lax.iota(jnp.int32, LANES)
    zero_col = jnp.zeros((LANES,), jnp.int32)
    one_col = jnp.full((LANES,), 1, jnp.int32)
    two_col = jnp.full((LANES,), 2, jnp.int32)

    def compute_block(i4, i2):
        cidx_b, nidx_b = cidx.at[i4], nidx.at[i4]
        posc_b, posn_b = posc.at[i2], posn.at[i2]
        outb_b = outb.at[i2]

        @plsc.parallel_loop(0, nvec, unroll=8)
        def compute_vec(j):
            jb = j * LANES
            rows = iota + jb
            cvec = cidx_b[pl.ds(jb, LANES)]
            nvec_i = nidx_b[pl.ds(jb, LANES)]
            tidc = plsc.load_gather(at_v, [cvec])
            tidn = plsc.load_gather(at_v, [nvec_i])

            cx = plsc.load_gather(posc_b, [rows, zero_col])
            cy = plsc.load_gather(posc_b, [rows, one_col])
            cz = plsc.load_gather(posc_b, [rows, two_col])
            nx = plsc.load_gather(posn_b, [rows, zero_col])
            ny = plsc.load_gather(posn_b, [rows, one_col])
            nz = plsc.load_gather(posn_b, [rows, two_col])

            dx = nx - cx
            dy = ny - cy
            dz = nz - cz
            s = dx * dx + dy * dy + dz * dz

            ibits = plsc.bitcast(s, jnp.int32)
            yi = jnp.full((LANES,), 0x5F3759DF, jnp.int32) - lax.shift_right_arithmetic(ibits, 1)
            y = plsc.bitcast(yi, jnp.float32)
            hs = 0.5 * s
            for _ in range(2):
                y = y * (1.5 - hs * y * y)
            x = s * y

            th = (PI / R_MAX) * x
            t2 = th * th
            s1 = _horner(SIN_C, t2) * th
            c1 = _horner(COS_C, t2)
            two_c1 = c1 + c1

            r = (1.0 / R_MAX) * x
            r2 = r * r
            r6 = r2 * r2 * r2
            cut = 1.0 - r6 * (28.0 - 48.0 * r + 21.0 * r2)
            cut = jnp.where(s < R_MAX * R_MAX, cut, 0.0)

            pcut = (PREF * cut) * y

            obase = (lax.shift_left(lax.shift_right_logical(rows, 7), 10)
                     + (rows & 127))

            def st(k, val):
                plsc.store_scatter(outb_b,
                                   [obase + ((k // 8) * 4096 + (k % 8) * 128)],
                                   val)

            st(0, s1 * pcut)
            skm2 = s1
            skm1 = two_c1 * s1
            st(1, skm1 * pcut)
            for k in range(3, 9):
                sk = two_c1 * skm1 - skm2
                st(k - 1, sk * pcut)
                skm2, skm1 = skm1, sk

            tc16 = tidc * 16
            tn16 = tidn * 16
            for c in range(16):
                v0 = plsc.load_gather(t0_v, [tc16 + c])
                st(8 + c, v0 * cut)
                v1 = plsc.load_gather(t1_v, [tn16 + c])
                st(24 + c, v1 * cut)

    for i in range(4):
        fire_idx(i, i)
    wait_idx(0)
    fire_gather(0, 0)

    n_iter = (nblk_w + 3) // 4

    def pipe_iter(t, _):
        s0 = t * 4
        for i in range(4):
            s = s0 + i
            i2 = i % 2
            wait_idx((i + 1) % 4)
            fire_gather((i + 1) % 4, (i + 1) % 2)
            wait_gather(i, i2)

            @pl.when((s - 2 >= 0) & (s - 2 < nblk_w))
            def _():
                wait_out(i2)

            compute_block(i, i2)

            @pl.when(s < nblk_w)
            def _():
                fire_out(s, i2)

            fire_idx(s + 4, i)
        return ()

    lax.fori_loop(0, n_iter, pipe_iter, (), unroll=False)

    wait_gather(0, 0)
    for i in (1, 2, 3):
        wait_idx(i)
    last = n_iter * 4

    @pl.when(last - 2 < nblk_w)
    def _():
        wait_out(0)

    @pl.when(last - 1 < nblk_w)
    def _():
        wait_out(1)


def kernel(pos, edge_index, atom_types, type_embeddings):
    n_atoms = pos.shape[0]
    n_edges = edge_index.shape[1]

    pos8 = jnp.pad(pos.astype(jnp.float32), ((0, 0), (0, 5)))
    cen = edge_index[0].astype(jnp.int32)
    nei = edge_index[1].astype(jnp.int32)
    at32 = atom_types.astype(jnp.int32)
    t0 = type_embeddings[0].astype(jnp.float32).reshape(-1)
    t1 = type_embeddings[1].astype(jnp.float32).reshape(-1)

    mesh = plsc.VectorSubcoreMesh(core_axis_name="c", subcore_axis_name="s",
                                  num_cores=2, num_subcores=16)
    body = functools.partial(_edge_body, n_atoms, n_edges)
    out = pl.kernel(
        body,
        out_type=jax.ShapeDtypeStruct((5, n_edges * 8), jnp.float32),
        mesh=mesh,
        compiler_params=pltpu.CompilerParams(needs_layout_passes=False,
                                             use_tc_tiling_on_sc=False),
        scratch_types=[
            pltpu.VMEM((n_atoms,), jnp.int32),
            pltpu.VMEM((256,), jnp.float32),
            pltpu.VMEM((256,), jnp.float32),
            pltpu.VMEM((4, BIG), jnp.int32),
            pltpu.VMEM((4, BIG), jnp.int32),
            pltpu.VMEM((2, BIG, 8), jnp.float32),
            pltpu.VMEM((2, BIG, 8), jnp.float32),
            pltpu.VMEM((2, BIG * 40), jnp.float32),
            pltpu.SemaphoreType.DMA((4,)),
            pltpu.SemaphoreType.DMA((2,)),
            pltpu.SemaphoreType.DMA((2,)),
        ],
    )(pos8, cen, nei, at32, t0, t1)
    return (out.reshape(5, n_edges // 128, 8, 128)
               .transpose(1, 3, 0, 2).reshape(n_edges, 40))
